# baseline (speedup 1.0000x reference)
import jax
import jax.numpy as jnp
from jax.experimental import pallas as pl
from jax.experimental.pallas import tpu as pltpu

ALPHA = 1.702
LIMIT = 7.0
FC = 512


def _moe_body(x_ref, w1_ref, w2_ref, rw_ref, b1_ref, b2_ref, out_ref, xb_ref):
    e = pl.program_id(0)

    @pl.when(e == 0)
    def _init():
        out_ref[...] = jnp.zeros_like(out_ref)
        xb_ref[...] = x_ref[...].astype(jnp.bfloat16)

    x = xb_ref[...]
    f = w2_ref.shape[1]
    rw_col = rw_ref[0, 0, :].reshape(-1, 1)
    for c in range(f // FC):
        sl = pl.ds(c * FC, FC)
        su = pl.ds(f + c * FC, FC)
        g = jnp.dot(x, w1_ref[0, :, sl], preferred_element_type=jnp.float32)
        u = jnp.dot(x, w1_ref[0, :, su], preferred_element_type=jnp.float32)
        g = g + b1_ref[0, 0, sl][None, :]
        u = u + b1_ref[0, 0, su][None, :]
        g = jnp.minimum(g, LIMIT)
        u = jnp.clip(u, -LIMIT, LIMIT)
        glu = g * jax.nn.sigmoid(g * ALPHA)
        h = ((u + 1.0) * glu * rw_col).astype(jnp.bfloat16)
        w2c = w2_ref[0, sl, :].astype(jnp.bfloat16)
        out_ref[...] += jnp.dot(h, w2c, preferred_element_type=jnp.float32)
    out_ref[...] += rw_col * b2_ref[0, 0, :][None, :]


@jax.jit
def kernel(hidden_states, router_indices, routing_weights, gate_up_proj,
           gate_up_proj_bias, down_proj, down_proj_bias):
    bsz, tt, hid = hidden_states.shape
    num_e, _, f2 = gate_up_proj.shape
    f = f2 // 2
    tok = bsz * tt

    x = hidden_states.reshape(tok, hid)
    w1 = jnp.transpose(gate_up_proj.reshape(num_e, hid, f, 2),
                       (0, 1, 3, 2)).reshape(num_e, hid, f2).astype(jnp.bfloat16)
    b1i = gate_up_proj_bias.reshape(num_e, f, 2)
    b1 = jnp.concatenate([b1i[..., 0], b1i[..., 1]], axis=-1).reshape(num_e, 1, f2)
    b2 = down_proj_bias.reshape(num_e, 1, hid)
    rw = routing_weights.T.reshape(num_e, 1, tok)

    out = pl.pallas_call(
        _moe_body,
        grid=(num_e,),
        in_specs=[
            pl.BlockSpec((tok, hid), lambda e: (0, 0)),
            pl.BlockSpec((1, hid, f2), lambda e: (e, 0, 0)),
            pl.BlockSpec((1, f, hid), lambda e: (e, 0, 0)),
            pl.BlockSpec((1, 1, tok), lambda e: (e, 0, 0)),
            pl.BlockSpec((1, 1, f2), lambda e: (e, 0, 0)),
            pl.BlockSpec((1, 1, hid), lambda e: (e, 0, 0)),
        ],
        out_specs=pl.BlockSpec((tok, hid), lambda e: (0, 0)),
        out_shape=jax.ShapeDtypeStruct((tok, hid), jnp.float32),
        scratch_shapes=[pltpu.VMEM((tok, hid), jnp.bfloat16)],
    )(x, w1, down_proj, rw, b1, b2)
    return out.reshape(bsz, tt, hid)


# speedup vs baseline: 1.0640x; 1.0640x over previous
import jax
import jax.numpy as jnp
from jax.experimental import pallas as pl
from jax.experimental.pallas import tpu as pltpu

ALPHA = 1.702
LIMIT = 7.0
FC = 512


def _moe_body(x_ref, w1_ref, w2_ref, rw_ref, b1_ref, b2_ref, out_ref, xb_ref):
    e = pl.program_id(0)

    @pl.when(e == 0)
    def _init():
        out_ref[...] = jnp.zeros_like(out_ref)
        xb_ref[...] = x_ref[...].astype(jnp.bfloat16)

    x = xb_ref[...]
    f = w2_ref.shape[1]
    rw_col = rw_ref[0, 0, :].reshape(-1, 1)
    for c in range(f // FC):
        sl = pl.ds(c * FC, FC)
        su = pl.ds(f + c * FC, FC)
        g = jnp.dot(x, w1_ref[0, :, sl], preferred_element_type=jnp.float32)
        u = jnp.dot(x, w1_ref[0, :, su], preferred_element_type=jnp.float32)
        g = g + b1_ref[0, 0, sl][None, :]
        u = u + b1_ref[0, 0, su][None, :]
        g = jnp.minimum(g, LIMIT)
        u = jnp.clip(u, -LIMIT, LIMIT)
        glu = g * jax.nn.sigmoid(g * ALPHA)
        h = ((u + 1.0) * glu * rw_col).astype(jnp.bfloat16)
        w2c = w2_ref[0, sl, :].astype(jnp.bfloat16)
        out_ref[...] += jnp.dot(h, w2c, preferred_element_type=jnp.float32)
    out_ref[...] += rw_col * b2_ref[0, 0, :][None, :]


@jax.jit
def kernel(hidden_states, router_indices, routing_weights, gate_up_proj,
           gate_up_proj_bias, down_proj, down_proj_bias):
    bsz, tt, hid = hidden_states.shape
    num_e, _, f2 = gate_up_proj.shape
    f = f2 // 2
    tok = bsz * tt

    x = hidden_states.reshape(tok, hid)
    gup = gate_up_proj.reshape(num_e, hid, f, 2)
    w1 = jnp.concatenate([gup[..., 0], gup[..., 1]], axis=-1).astype(jnp.bfloat16)
    b1i = gate_up_proj_bias.reshape(num_e, f, 2)
    b1 = jnp.concatenate([b1i[..., 0], b1i[..., 1]], axis=-1).reshape(num_e, 1, f2)
    b2 = down_proj_bias.reshape(num_e, 1, hid)
    rw = routing_weights.T.reshape(num_e, 1, tok)

    out = pl.pallas_call(
        _moe_body,
        grid=(num_e,),
        in_specs=[
            pl.BlockSpec((tok, hid), lambda e: (0, 0)),
            pl.BlockSpec((1, hid, f2), lambda e: (e, 0, 0)),
            pl.BlockSpec((1, f, hid), lambda e: (e, 0, 0)),
            pl.BlockSpec((1, 1, tok), lambda e: (e, 0, 0)),
            pl.BlockSpec((1, 1, f2), lambda e: (e, 0, 0)),
            pl.BlockSpec((1, 1, hid), lambda e: (e, 0, 0)),
        ],
        out_specs=pl.BlockSpec((tok, hid), lambda e: (0, 0)),
        out_shape=jax.ShapeDtypeStruct((tok, hid), jnp.float32),
        scratch_shapes=[pltpu.VMEM((tok, hid), jnp.bfloat16)],
    )(x, w1, down_proj, rw, b1, b2)
    return out.reshape(bsz, tt, hid)


# speedup vs baseline: 1.0643x; 1.0003x over previous
import jax
import jax.numpy as jnp
from jax.experimental import pallas as pl
from jax.experimental.pallas import tpu as pltpu

ALPHA = 1.702
LIMIT = 7.0
FC = 512


def _moe_body(x_ref, w1_ref, w2_ref, rw_ref, b1_ref, b2_ref, out_ref, xb_ref):
    e = pl.program_id(0)

    @pl.when(e == 0)
    def _init():
        out_ref[...] = jnp.zeros_like(out_ref)
        xb_ref[...] = x_ref[...].astype(jnp.bfloat16)

    x = xb_ref[...]
    f = w2_ref.shape[1]
    rw_col = rw_ref[0, 0, :].reshape(-1, 1)
    for c in range(f // FC):
        sl = pl.ds(c * FC, FC)
        su = pl.ds(f + c * FC, FC)
        g = jnp.dot(x, w1_ref[0, :, sl], preferred_element_type=jnp.float32)
        u = jnp.dot(x, w1_ref[0, :, su], preferred_element_type=jnp.float32)
        g = g + b1_ref[0, 0, sl][None, :]
        u = u + b1_ref[0, 0, su][None, :]
        g = jnp.minimum(g, LIMIT)
        u = jnp.clip(u, -LIMIT, LIMIT)
        glu = g * jax.nn.sigmoid(g * ALPHA)
        h = ((u + 1.0) * glu * rw_col).astype(jnp.bfloat16)
        w2c = w2_ref[0, sl, :].astype(jnp.bfloat16)
        out_ref[...] += jnp.dot(h, w2c, preferred_element_type=jnp.float32)
    out_ref[...] += rw_col * b2_ref[0, 0, :][None, :]


@jax.jit
def kernel(hidden_states, router_indices, routing_weights, gate_up_proj,
           gate_up_proj_bias, down_proj, down_proj_bias):
    bsz, tt, hid = hidden_states.shape
    num_e, _, f2 = gate_up_proj.shape
    f = f2 // 2
    tok = bsz * tt

    x = hidden_states.reshape(tok, hid)
    gup = gate_up_proj.reshape(num_e, hid, f, 2)
    w1f = jnp.concatenate([gup[..., 0], gup[..., 1]], axis=-1)
    w1 = jax.lax.optimization_barrier(w1f).astype(jnp.bfloat16)
    b1i = gate_up_proj_bias.reshape(num_e, f, 2)
    b1 = jnp.concatenate([b1i[..., 0], b1i[..., 1]], axis=-1).reshape(num_e, 1, f2)
    b2 = down_proj_bias.reshape(num_e, 1, hid)
    rw = routing_weights.T.reshape(num_e, 1, tok)

    out = pl.pallas_call(
        _moe_body,
        grid=(num_e,),
        in_specs=[
            pl.BlockSpec((tok, hid), lambda e: (0, 0)),
            pl.BlockSpec((1, hid, f2), lambda e: (e, 0, 0)),
            pl.BlockSpec((1, f, hid), lambda e: (e, 0, 0)),
            pl.BlockSpec((1, 1, tok), lambda e: (e, 0, 0)),
            pl.BlockSpec((1, 1, f2), lambda e: (e, 0, 0)),
            pl.BlockSpec((1, 1, hid), lambda e: (e, 0, 0)),
        ],
        out_specs=pl.BlockSpec((tok, hid), lambda e: (0, 0)),
        out_shape=jax.ShapeDtypeStruct((tok, hid), jnp.float32),
        scratch_shapes=[pltpu.VMEM((tok, hid), jnp.bfloat16)],
    )(x, w1, down_proj, rw, b1, b2)
    return out.reshape(bsz, tt, hid)
